# baseline (speedup 1.0000x reference)
import jax
import jax.numpy as jnp
from jax.experimental import pallas as pl
from jax.experimental.pallas import tpu as pltpu

_NCHUNK = 4


def _dma_copy(x_hbm, o_hbm, *sems):
    rows = x_hbm.shape[0]
    chunk = rows // _NCHUNK
    copies = [
        pltpu.make_async_copy(
            x_hbm.at[pl.ds(i * chunk, chunk), :],
            o_hbm.at[pl.ds(i * chunk, chunk), :],
            sems[i],
        )
        for i in range(_NCHUNK)
    ]
    for c in copies:
        c.start()
    for c in copies:
        c.wait()


def kernel(features, labels):
    del labels
    n, h, w = features.shape
    rows, cols = n * h, w
    flat = features.reshape(rows, cols)
    out = pl.pallas_call(
        _dma_copy,
        in_specs=[pl.BlockSpec(memory_space=pl.ANY)],
        out_specs=pl.BlockSpec(memory_space=pl.ANY),
        out_shape=jax.ShapeDtypeStruct((rows, cols), features.dtype),
        scratch_shapes=[pltpu.SemaphoreType.DMA] * _NCHUNK,
    )(flat)
    return out.reshape(n, h, w)
